# baseline (speedup 1.0000x reference)
import functools

import jax
import jax.numpy as jnp
from jax import lax
from jax.experimental import pallas as pl
from jax.experimental.pallas import tpu as pltpu
from jax.experimental.pallas import tpu_sc as plsc

VOCAB = 100000
DIM = 64
B = 1024
MAX_NORM = 1.0

_NC = 2
_NS = 16
_NW = _NC * _NS
_B_PER_W = B // _NW

_N_BLK = 2048
_N_GRID = (VOCAB + _N_BLK - 1) // _N_BLK


@functools.lru_cache(maxsize=1)
def _make_sc_gather():
    @functools.partial(
        pl.kernel,
        mesh=plsc.VectorSubcoreMesh(core_axis_name="c", subcore_axis_name="s"),
        out_type=jax.ShapeDtypeStruct((B, DIM), jnp.float32),
        scratch_types=[
            pltpu.VMEM((_B_PER_W,), jnp.int32),
            pltpu.VMEM((_B_PER_W, DIM), jnp.float32),
            pltpu.SemaphoreType.DMA,
        ],
        compiler_params=pltpu.CompilerParams(use_tc_tiling_on_sc=False),
    )
    def _sc_gather(idx_hbm, table_hbm, out_hbm, idx_v, rows_v, sem):
        wid = lax.axis_index("s") * _NC + lax.axis_index("c")
        base = wid * _B_PER_W
        pltpu.sync_copy(idx_hbm.at[pl.ds(base, _B_PER_W)], idx_v)
        pltpu.async_copy(table_hbm.at[idx_v], rows_v, sem).wait()
        pltpu.sync_copy(rows_v, out_hbm.at[pl.ds(base, _B_PER_W)])

    return _sc_gather


def _mm_body(emb_ref, w_ref, b_ref, out_ref):
    emb = emb_ref[...]
    ss = jnp.sum(emb * emb, axis=1, keepdims=True)
    scale = jnp.where(
        ss > MAX_NORM * MAX_NORM, MAX_NORM / (jnp.sqrt(ss) + 1e-7), 1.0
    )
    emb = emb * scale
    out_ref[...] = (
        lax.dot_general(
            emb,
            w_ref[...],
            (((1,), (1,)), ((), ())),
            preferred_element_type=jnp.float32,
        )
        + b_ref[...]
    )


def _tc_project(emb, W, b2d):
    return pl.pallas_call(
        _mm_body,
        grid=(_N_GRID,),
        in_specs=[
            pl.BlockSpec((B, DIM), lambda j: (0, 0)),
            pl.BlockSpec((_N_BLK, DIM), lambda j: (j, 0)),
            pl.BlockSpec((1, _N_BLK), lambda j: (0, j)),
        ],
        out_specs=pl.BlockSpec((B, _N_BLK), lambda j: (0, j)),
        out_shape=jax.ShapeDtypeStruct((B, VOCAB), jnp.float32),
    )(emb, W, b2d)


@jax.jit
def kernel(inputs_, table, W, b):
    idx = inputs_.astype(jnp.int32)
    emb = _make_sc_gather()(idx, table)
    return _tc_project(emb, W, b.reshape(1, VOCAB))


# speedup vs baseline: 2.9086x; 2.9086x over previous
import functools

import jax
import jax.numpy as jnp
from jax import lax
from jax.experimental import pallas as pl
from jax.experimental.pallas import tpu as pltpu
from jax.experimental.pallas import tpu_sc as plsc

VOCAB = 100000
DIM = 64
DPAD = 128
B = 1024
MAX_NORM = 1.0

_NC = 2
_NS = 16
_NW = _NC * _NS
_B_PER_W = B // _NW

_N_BLK = 2048
_N_GRID = (VOCAB + _N_BLK - 1) // _N_BLK


@functools.lru_cache(maxsize=1)
def _make_sc_gather():
    @functools.partial(
        pl.kernel,
        mesh=plsc.VectorSubcoreMesh(core_axis_name="c", subcore_axis_name="s"),
        out_type=jax.ShapeDtypeStruct((B, DPAD), jnp.float32),
        scratch_types=[
            pltpu.VMEM((_B_PER_W,), jnp.int32),
            pltpu.VMEM((_B_PER_W, DPAD), jnp.float32),
            pltpu.SemaphoreType.DMA,
        ],
    )
    def _sc_gather(idx_hbm, table_hbm, out_hbm, idx_v, rows_v, sem):
        wid = lax.axis_index("s") * _NC + lax.axis_index("c")
        base = wid * _B_PER_W
        pltpu.sync_copy(idx_hbm.at[pl.ds(base, _B_PER_W)], idx_v)
        pltpu.async_copy(table_hbm.at[idx_v], rows_v, sem).wait()
        pltpu.sync_copy(rows_v, out_hbm.at[pl.ds(base, _B_PER_W)])

    return _sc_gather


def _mm_body(emb_ref, wt_ref, b_ref, out_ref):
    emb = emb_ref[:, :DIM]
    ss = jnp.sum(emb * emb, axis=1, keepdims=True)
    scale = jnp.where(
        ss > MAX_NORM * MAX_NORM, MAX_NORM / (jnp.sqrt(ss) + 1e-7), 1.0
    )
    emb = emb * scale
    acc = lax.dot_general(
        wt_ref[...],
        emb,
        (((0,), (1,)), ((), ())),
        preferred_element_type=jnp.float32,
    )
    ones = jnp.ones((1, B), dtype=jnp.float32)
    bias = lax.dot_general(
        b_ref[...],
        ones,
        (((0,), (0,)), ((), ())),
        preferred_element_type=jnp.float32,
    )
    out_ref[...] = acc + bias


def _tc_project(emb, wT, b2d):
    return pl.pallas_call(
        _mm_body,
        grid=(_N_GRID,),
        in_specs=[
            pl.BlockSpec((B, DPAD), lambda j: (0, 0)),
            pl.BlockSpec((DIM, _N_BLK), lambda j: (0, j)),
            pl.BlockSpec((1, _N_BLK), lambda j: (0, j)),
        ],
        out_specs=pl.BlockSpec((_N_BLK, B), lambda j: (j, 0)),
        out_shape=jax.ShapeDtypeStruct((VOCAB, B), jnp.float32),
    )(emb, wT, b2d)


@jax.jit
def kernel(inputs_, table, W, b):
    idx = inputs_.astype(jnp.int32)
    tpad = jnp.pad(table, ((0, 0), (0, DPAD - DIM)))
    emb = _make_sc_gather()(idx, tpad)
    out_t = _tc_project(emb, W.T, b.reshape(1, VOCAB))
    return out_t.T
